# Initial kernel scaffold; baseline (speedup 1.0000x reference)
import jax
import jax.numpy as jnp
from jax.experimental import pallas as pl


def kernel(h_src, h_dst, e_h, edge_index, params):
    raise NotImplementedError("write your pallas kernel here")


# speedup vs baseline: 1.1531x; 1.1531x over previous
import functools

import jax
import jax.numpy as jnp
from jax import lax
from jax.experimental import pallas as pl
from jax.experimental.pallas import tpu as pltpu
from jax.experimental.pallas import tpu_sc as plsc

NC = 2
NS = 16
NW = NC * NS
CHUNK = 400
NPAD = 10240


def _sc_mesh():
    return plsc.VectorSubcoreMesh(
        core_axis_name="c", subcore_axis_name="s", num_cores=NC, num_subcores=NS
    )


def _sc_gather(h_src, h_dst, src, dst):
    N, D = h_src.shape
    E = src.shape[0]
    EW = E // NW
    C = CHUNK

    @functools.partial(
        pl.kernel,
        out_type=[
            jax.ShapeDtypeStruct((E, D), jnp.float32),
            jax.ShapeDtypeStruct((E, D), jnp.float32),
        ],
        mesh=_sc_mesh(),
        scratch_types=[
            pltpu.VMEM((C,), jnp.int32),
            pltpu.VMEM((C,), jnp.int32),
            pltpu.VMEM((C, D), jnp.float32),
            pltpu.VMEM((C, D), jnp.float32),
            pltpu.SemaphoreType.DMA,
            pltpu.SemaphoreType.DMA,
        ],
    )
    def gather_kernel(hs_hbm, hd_hbm, src_hbm, dst_hbm, out_s, out_d,
                      idx_s, idx_d, rows_s, rows_d, sem_s, sem_d):
        wid = lax.axis_index("s") * NC + lax.axis_index("c")
        base = wid * EW

        def body(i, carry):
            off = base + i * C
            pltpu.sync_copy(src_hbm.at[pl.ds(off, C)], idx_s)
            pltpu.sync_copy(dst_hbm.at[pl.ds(off, C)], idx_d)
            cs = pltpu.async_copy(hs_hbm.at[idx_s], rows_s, sem_s)
            cd = pltpu.async_copy(hd_hbm.at[idx_d], rows_d, sem_d)
            cs.wait()
            cd.wait()
            pltpu.sync_copy(rows_s, out_s.at[pl.ds(off, C)])
            pltpu.sync_copy(rows_d, out_d.at[pl.ds(off, C)])
            return carry

        lax.fori_loop(0, EW // C, body, 0)

    return gather_kernel(h_src, h_dst, src, dst)


HALF = 5120
ACC_ROWS = 5248


def _sc_scatter(e_new, dst, zeros_pad):
    E, D = e_new.shape
    ET = E // NS
    C = CHUNK
    STRIPE = HALF // NS

    @functools.partial(
        pl.kernel,
        out_type=jax.ShapeDtypeStruct((NC * HALF, D), jnp.float32),
        mesh=_sc_mesh(),
        scratch_types=[
            pltpu.VMEM((C,), jnp.int32),
            pltpu.VMEM((C,), jnp.int32),
            pltpu.VMEM((C, D), jnp.float32),
            pltpu.VMEM_SHARED((ACC_ROWS, D), jnp.float32),
            pltpu.SemaphoreType.DMA,
        ],
    )
    def scatter_kernel(enew_hbm, dst_hbm, zeros_hbm, acc_out,
                       idx_v, idx2_v, rows_v, acc_sh, sem):
        cid = lax.axis_index("c")
        sid = lax.axis_index("s")
        zst = ACC_ROWS // NS
        pltpu.sync_copy(zeros_hbm.at[pl.ds(sid * zst, zst)],
                        acc_sh.at[pl.ds(sid * zst, zst)])
        plsc.subcore_barrier()
        base = sid * ET
        lo = cid * HALF

        def body(i, carry):
            off = base + i * C
            pltpu.sync_copy(dst_hbm.at[pl.ds(off, C)], idx_v)
            pltpu.sync_copy(enew_hbm.at[pl.ds(off, C)], rows_v)

            def remap(j, carry2):
                v = idx_v[pl.ds(j * 16, 16)] - lo
                ok = (v >= 0) & (v < HALF)
                idx2_v[pl.ds(j * 16, 16)] = jnp.where(ok, v, HALF)
                return carry2

            lax.fori_loop(0, C // 16, remap, 0)
            pltpu.sync_copy(rows_v, acc_sh.at[idx2_v], add=True)
            return carry

        lax.fori_loop(0, ET // C, body, 0)
        plsc.subcore_barrier()
        pltpu.sync_copy(acc_sh.at[pl.ds(sid * STRIPE, STRIPE)],
                        acc_out.at[pl.ds(cid * HALF + sid * STRIPE, STRIPE)])

    return scatter_kernel(e_new, dst, zeros_pad)


def _ln(x, g, b):
    m = jnp.mean(x, axis=-1, keepdims=True)
    v = jnp.mean((x - m) ** 2, axis=-1, keepdims=True)
    return (x - m) * lax.rsqrt(v + 1e-5) * g + b


def _dot(a, b):
    return jnp.dot(a, b, preferred_element_type=jnp.float32,
                   precision=lax.Precision.HIGHEST)


def _edge_call(e_h, hs_g, hd_g, p):
    E, D = e_h.shape
    B = 2000
    G = E // B
    W1 = p['W1']
    w1e, w1s, w1d = W1[:D], W1[D:2 * D], W1[2 * D:]
    H = W1.shape[0]
    vec = lambda v: v.reshape(1, -1)

    def body(eh, hs, hd, rw1e, rw1s, rw1d, rb1, rg1, rbe1, rw2, rb2, rg2, rbe2,
             eout, enew):
        x = eh[...]
        h = (_dot(x, rw1e[...]) + _dot(hs[...], rw1s[...])
             + _dot(hd[...], rw1d[...]) + rb1[...])
        h = _ln(h, rg1[...], rbe1[...])
        h = h * jax.nn.sigmoid(h)
        o = _dot(h, rw2[...]) + rb2[...]
        o = _ln(o, rg2[...], rbe2[...])
        enew[...] = o
        eout[...] = x + o

    row = pl.BlockSpec((B, D), lambda i: (i, 0))
    full = lambda a: pl.BlockSpec(a.shape, lambda i: (0,) * a.ndim)
    args = [w1e, w1s, w1d, vec(p['b1']), vec(p['g1']), vec(p['be1']),
            p['W2'], vec(p['b2']), vec(p['g2']), vec(p['be2'])]
    return pl.pallas_call(
        body,
        grid=(G,),
        in_specs=[row, row, row] + [full(a) for a in args],
        out_specs=[row, row],
        out_shape=[jax.ShapeDtypeStruct((E, D), jnp.float32),
                   jax.ShapeDtypeStruct((E, D), jnp.float32)],
    )(e_h, hs_g, hd_g, *args)


def _node_call(h_src, h_dst, m, ps, pd):
    N, D = h_src.shape
    B = 1000
    G = N // B
    wd1h, wd1m = pd['W1'][:D], pd['W1'][D:]
    vec = lambda v: v.reshape(1, -1)

    def body(hs, hd, mr,
             sw1, sb1, sg1, sbe1, sw2, sb2, sg2, sbe2,
             dw1h, dw1m, db1, dg1, dbe1, dw2, db2, dg2, dbe2,
             hs_out, hd_out):
        x = hs[...]
        h = _dot(x, sw1[...]) + sb1[...]
        h = _ln(h, sg1[...], sbe1[...])
        h = h * jax.nn.sigmoid(h)
        o = _dot(h, sw2[...]) + sb2[...]
        o = _ln(o, sg2[...], sbe2[...])
        hs_out[...] = x + o

        xd = hd[...]
        h2 = _dot(xd, dw1h[...]) + _dot(mr[...], dw1m[...]) + db1[...]
        h2 = _ln(h2, dg1[...], dbe1[...])
        h2 = h2 * jax.nn.sigmoid(h2)
        o2 = _dot(h2, dw2[...]) + db2[...]
        o2 = _ln(o2, dg2[...], dbe2[...])
        hd_out[...] = xd + o2

    row = pl.BlockSpec((B, D), lambda i: (i, 0))
    full = lambda a: pl.BlockSpec(a.shape, lambda i: (0,) * a.ndim)
    args = [ps['W1'], vec(ps['b1']), vec(ps['g1']), vec(ps['be1']),
            ps['W2'], vec(ps['b2']), vec(ps['g2']), vec(ps['be2']),
            wd1h, wd1m, vec(pd['b1']), vec(pd['g1']), vec(pd['be1']),
            pd['W2'], vec(pd['b2']), vec(pd['g2']), vec(pd['be2'])]
    return pl.pallas_call(
        body,
        grid=(G,),
        in_specs=[row, row, row] + [full(a) for a in args],
        out_specs=[row, row],
        out_shape=[jax.ShapeDtypeStruct((N, D), jnp.float32),
                   jax.ShapeDtypeStruct((N, D), jnp.float32)],
    )(h_src, h_dst, m, *args)


def kernel(h_src, h_dst, e_h, edge_index, params):
    src = edge_index[0].astype(jnp.int32)
    dst = edge_index[1].astype(jnp.int32)
    hs_g, hd_g = _sc_gather(h_src, h_dst, src, dst)
    e_out, e_new = _edge_call(e_h, hs_g, hd_g, params['edge'])
    zeros = jnp.zeros((ACC_ROWS, e_h.shape[1]), jnp.float32)
    m = _sc_scatter(e_new, dst, zeros)
    h_src_out, h_dst_out = _node_call(h_src, h_dst, m,
                                      params['src'], params['dst'])
    return (h_src_out, h_dst_out, e_out)


# speedup vs baseline: 3.1028x; 2.6908x over previous
import functools

import jax
import jax.numpy as jnp
from jax import lax
from jax.experimental import pallas as pl
from jax.experimental.pallas import tpu as pltpu
from jax.experimental.pallas import tpu_sc as plsc

NC = 2
NS = 16
NW = NC * NS
CHUNK = 400
NPAD = 10240


def _sc_mesh():
    return plsc.VectorSubcoreMesh(
        core_axis_name="c", subcore_axis_name="s", num_cores=NC, num_subcores=NS
    )


def _sc_gather(h_src, h_dst, src, dst):
    N, D = h_src.shape
    E = src.shape[0]
    EW = E // NW
    C = CHUNK

    @functools.partial(
        pl.kernel,
        out_type=[
            jax.ShapeDtypeStruct((E, D), jnp.float32),
            jax.ShapeDtypeStruct((E, D), jnp.float32),
        ],
        mesh=_sc_mesh(),
        scratch_types=[
            pltpu.VMEM((C,), jnp.int32),
            pltpu.VMEM((C,), jnp.int32),
            pltpu.VMEM((C, D), jnp.float32),
            pltpu.VMEM((C, D), jnp.float32),
            pltpu.SemaphoreType.DMA,
            pltpu.SemaphoreType.DMA,
        ],
    )
    def gather_kernel(hs_hbm, hd_hbm, src_hbm, dst_hbm, out_s, out_d,
                      idx_s, idx_d, rows_s, rows_d, sem_s, sem_d):
        wid = lax.axis_index("s") * NC + lax.axis_index("c")
        base = wid * EW

        def body(i, carry):
            off = base + i * C
            pltpu.sync_copy(src_hbm.at[pl.ds(off, C)], idx_s)
            pltpu.sync_copy(dst_hbm.at[pl.ds(off, C)], idx_d)
            cs = pltpu.async_copy(hs_hbm.at[idx_s], rows_s, sem_s)
            cd = pltpu.async_copy(hd_hbm.at[idx_d], rows_d, sem_d)
            cs.wait()
            cd.wait()
            pltpu.sync_copy(rows_s, out_s.at[pl.ds(off, C)])
            pltpu.sync_copy(rows_d, out_d.at[pl.ds(off, C)])
            return carry

        lax.fori_loop(0, EW // C, body, 0)

    return gather_kernel(h_src, h_dst, src, dst)


HALF = 5120
ACC_ROWS = 5248


def _sc_scatter(e_new, dst, zeros_pad):
    E, D = e_new.shape
    ET = E // NS
    C = CHUNK
    STRIPE = HALF // NS

    @functools.partial(
        pl.kernel,
        out_type=jax.ShapeDtypeStruct((NC * HALF, D), jnp.float32),
        mesh=_sc_mesh(),
        scratch_types=[
            pltpu.VMEM((C,), jnp.int32),
            pltpu.VMEM((C,), jnp.int32),
            pltpu.VMEM((C, D), jnp.float32),
            pltpu.VMEM_SHARED((ACC_ROWS, D), jnp.float32),
            pltpu.SemaphoreType.DMA,
        ],
    )
    def scatter_kernel(enew_hbm, dst_hbm, zeros_hbm, acc_out,
                       idx_v, idx2_v, rows_v, acc_sh, sem):
        cid = lax.axis_index("c")
        sid = lax.axis_index("s")
        zst = ACC_ROWS // NS
        pltpu.sync_copy(zeros_hbm.at[pl.ds(sid * zst, zst)],
                        acc_sh.at[pl.ds(sid * zst, zst)])
        plsc.subcore_barrier()
        base = sid * ET
        lo = cid * HALF

        def body(i, carry):
            off = base + i * C
            pltpu.sync_copy(dst_hbm.at[pl.ds(off, C)], idx_v)
            pltpu.sync_copy(enew_hbm.at[pl.ds(off, C)], rows_v)

            def remap(j, carry2):
                v = idx_v[pl.ds(j * 16, 16)] - lo
                ok = (v >= 0) & (v < HALF)
                idx2_v[pl.ds(j * 16, 16)] = jnp.where(ok, v, HALF)
                return carry2

            lax.fori_loop(0, C // 16, remap, 0)
            pltpu.sync_copy(rows_v, acc_sh.at[idx2_v], add=True)
            return carry

        lax.fori_loop(0, ET // C, body, 0)
        plsc.subcore_barrier()
        pltpu.sync_copy(acc_sh.at[pl.ds(sid * STRIPE, STRIPE)],
                        acc_out.at[pl.ds(cid * HALF + sid * STRIPE, STRIPE)])

    return scatter_kernel(e_new, dst, zeros_pad)


def _ln(x, g, b):
    m = jnp.mean(x, axis=-1, keepdims=True)
    v = jnp.mean((x - m) ** 2, axis=-1, keepdims=True)
    return (x - m) * lax.rsqrt(v + 1e-5) * g + b


def _dot(a, b):
    return jnp.dot(a, b, preferred_element_type=jnp.float32,
                   precision=lax.Precision.DEFAULT)


def _edge_call(e_h, hs_g, hd_g, p):
    E, D = e_h.shape
    B = 2000
    G = E // B
    W1 = p['W1']
    w1e, w1s, w1d = W1[:D], W1[D:2 * D], W1[2 * D:]
    H = W1.shape[0]
    vec = lambda v: v.reshape(1, -1)

    def body(eh, hs, hd, rw1e, rw1s, rw1d, rb1, rg1, rbe1, rw2, rb2, rg2, rbe2,
             eout, enew):
        x = eh[...]
        h = (_dot(x, rw1e[...]) + _dot(hs[...], rw1s[...])
             + _dot(hd[...], rw1d[...]) + rb1[...])
        h = _ln(h, rg1[...], rbe1[...])
        h = h * jax.nn.sigmoid(h)
        o = _dot(h, rw2[...]) + rb2[...]
        o = _ln(o, rg2[...], rbe2[...])
        enew[...] = o
        eout[...] = x + o

    row = pl.BlockSpec((B, D), lambda i: (i, 0))
    full = lambda a: pl.BlockSpec(a.shape, lambda i: (0,) * a.ndim)
    args = [w1e, w1s, w1d, vec(p['b1']), vec(p['g1']), vec(p['be1']),
            p['W2'], vec(p['b2']), vec(p['g2']), vec(p['be2'])]
    return pl.pallas_call(
        body,
        grid=(G,),
        in_specs=[row, row, row] + [full(a) for a in args],
        out_specs=[row, row],
        out_shape=[jax.ShapeDtypeStruct((E, D), jnp.float32),
                   jax.ShapeDtypeStruct((E, D), jnp.float32)],
    )(e_h, hs_g, hd_g, *args)


def _node_call(h_src, h_dst, m, ps, pd):
    N, D = h_src.shape
    B = 1000
    G = N // B
    wd1h, wd1m = pd['W1'][:D], pd['W1'][D:]
    vec = lambda v: v.reshape(1, -1)

    def body(hs, hd, mr,
             sw1, sb1, sg1, sbe1, sw2, sb2, sg2, sbe2,
             dw1h, dw1m, db1, dg1, dbe1, dw2, db2, dg2, dbe2,
             hs_out, hd_out):
        x = hs[...]
        h = _dot(x, sw1[...]) + sb1[...]
        h = _ln(h, sg1[...], sbe1[...])
        h = h * jax.nn.sigmoid(h)
        o = _dot(h, sw2[...]) + sb2[...]
        o = _ln(o, sg2[...], sbe2[...])
        hs_out[...] = x + o

        xd = hd[...]
        h2 = _dot(xd, dw1h[...]) + _dot(mr[...], dw1m[...]) + db1[...]
        h2 = _ln(h2, dg1[...], dbe1[...])
        h2 = h2 * jax.nn.sigmoid(h2)
        o2 = _dot(h2, dw2[...]) + db2[...]
        o2 = _ln(o2, dg2[...], dbe2[...])
        hd_out[...] = xd + o2

    row = pl.BlockSpec((B, D), lambda i: (i, 0))
    full = lambda a: pl.BlockSpec(a.shape, lambda i: (0,) * a.ndim)
    args = [ps['W1'], vec(ps['b1']), vec(ps['g1']), vec(ps['be1']),
            ps['W2'], vec(ps['b2']), vec(ps['g2']), vec(ps['be2']),
            wd1h, wd1m, vec(pd['b1']), vec(pd['g1']), vec(pd['be1']),
            pd['W2'], vec(pd['b2']), vec(pd['g2']), vec(pd['be2'])]
    return pl.pallas_call(
        body,
        grid=(G,),
        in_specs=[row, row, row] + [full(a) for a in args],
        out_specs=[row, row],
        out_shape=[jax.ShapeDtypeStruct((N, D), jnp.float32),
                   jax.ShapeDtypeStruct((N, D), jnp.float32)],
    )(h_src, h_dst, m, *args)


def kernel(h_src, h_dst, e_h, edge_index, params):
    src = edge_index[0].astype(jnp.int32)
    dst = edge_index[1].astype(jnp.int32)
    hs_g, hd_g = _sc_gather(h_src, h_dst, src, dst)
    e_out, e_new = _edge_call(e_h, hs_g, hd_g, params['edge'])
    zeros = jnp.zeros((ACC_ROWS, e_h.shape[1]), jnp.float32)
    m = _sc_scatter(e_new, dst, zeros)
    h_src_out, h_dst_out = _node_call(h_src, h_dst, m,
                                      params['src'], params['dst'])
    return (h_src_out, h_dst_out, e_out)


# speedup vs baseline: 3.1983x; 1.0308x over previous
import functools

import jax
import jax.numpy as jnp
from jax import lax
from jax.experimental import pallas as pl
from jax.experimental.pallas import tpu as pltpu
from jax.experimental.pallas import tpu_sc as plsc

NC = 2
NS = 16
NW = NC * NS
CHUNK = 400
NPAD = 10240


def _sc_mesh():
    return plsc.VectorSubcoreMesh(
        core_axis_name="c", subcore_axis_name="s", num_cores=NC, num_subcores=NS
    )


def _sc_gather(h_src, h_dst, src, dst):
    N, D = h_src.shape
    E = src.shape[0]
    EW = E // NW
    C = 200
    NCH = EW // C

    @functools.partial(
        pl.kernel,
        out_type=[jax.ShapeDtypeStruct((E, D), jnp.float32),
                  jax.ShapeDtypeStruct((E, D), jnp.float32)],
        mesh=_sc_mesh(),
        scratch_types=[
            pltpu.VMEM((C,), jnp.int32),
            pltpu.VMEM((C,), jnp.int32),
            pltpu.VMEM((C,), jnp.int32),
            pltpu.VMEM((C,), jnp.int32),
            pltpu.VMEM((C, D), jnp.float32),
            pltpu.VMEM((C, D), jnp.float32),
            pltpu.VMEM((C, D), jnp.float32),
            pltpu.VMEM((C, D), jnp.float32),
            pltpu.SemaphoreType.DMA((12,)),
        ],
    )
    def gather_kernel(hs_hbm, hd_hbm, src_hbm, dst_hbm, out_s, out_d,
                      ibs0, ibs1, ibd0, ibd1, rs0, rs1, rd0, rd1, sem):
        wid = lax.axis_index("s") * NC + lax.axis_index("c")
        base = wid * EW
        ibs = (ibs0, ibs1)
        ibd = (ibd0, ibd1)
        rs = (rs0, rs1)
        rd = (rd0, rd1)
        si_s = (sem.at[0], sem.at[1])
        si_d = (sem.at[2], sem.at[3])
        sg_s = (sem.at[4], sem.at[5])
        sg_d = (sem.at[6], sem.at[7])
        sw_s = (sem.at[8], sem.at[9])
        sw_d = (sem.at[10], sem.at[11])

        def w_start(i, b):
            off = base + i * C
            cs = pltpu.async_copy(rs[b], out_s.at[pl.ds(off, C)], sw_s[b])
            cd = pltpu.async_copy(rd[b], out_d.at[pl.ds(off, C)], sw_d[b])
            return (cs, cd)

        def wait_all(cs):
            for c in cs:
                c.wait()

        def ix_start(i, b):
            off = base + i * C
            cs = pltpu.async_copy(src_hbm.at[pl.ds(off, C)], ibs[b], si_s[b])
            cd = pltpu.async_copy(dst_hbm.at[pl.ds(off, C)], ibd[b], si_d[b])
            return (cs, cd)

        def g_start(b):
            cs = pltpu.async_copy(hs_hbm.at[ibs[b]], rs[b], sg_s[b])
            cd = pltpu.async_copy(hd_hbm.at[ibd[b]], rd[b], sg_d[b])
            return (cs, cd)

        wait_all(ix_start(0, 0))
        wait_all(ix_start(1, 1))
        g_pend = [g_start(0), g_start(1)]
        w_pend = [None, None]
        for i in range(NCH):
            b = i % 2
            wait_all(g_pend[b])
            w_pend[b] = w_start(i, b)
            if i + 2 < NCH:
                ix = ix_start(i + 2, b)
                wait_all(w_pend[b])
                wait_all(ix)
                g_pend[b] = g_start(b)
        wait_all(w_pend[0])
        wait_all(w_pend[1])

    return gather_kernel(h_src, h_dst, src, dst)


HALF = 5120
ACC_ROWS = 5248


def _sc_scatter(e_new, dst, zeros_pad):
    E, D = e_new.shape
    ET = E // NS
    C = 400
    NCH = ET // C
    STRIPE = HALF // NS

    @functools.partial(
        pl.kernel,
        out_type=jax.ShapeDtypeStruct((NC * HALF, D), jnp.float32),
        mesh=_sc_mesh(),
        scratch_types=[
            pltpu.VMEM((C,), jnp.int32),
            pltpu.VMEM((C,), jnp.int32),
            pltpu.VMEM((C, D), jnp.float32),
            pltpu.VMEM_SHARED((ACC_ROWS, D), jnp.float32),
            pltpu.SemaphoreType.DMA,
        ],
    )
    def scatter_kernel(enew_hbm, dst_hbm, zeros_hbm, acc_out,
                       idx_v, idx2_v, rows_v, acc_sh, sem):
        cid = lax.axis_index("c")
        sid = lax.axis_index("s")
        zst = ACC_ROWS // NS
        pltpu.sync_copy(zeros_hbm.at[pl.ds(sid * zst, zst)],
                        acc_sh.at[pl.ds(sid * zst, zst)])
        plsc.subcore_barrier()
        base = sid * ET
        lo = cid * HALF

        def body(i, carry):
            off = base + i * C
            pltpu.sync_copy(dst_hbm.at[pl.ds(off, C)], idx_v)
            pltpu.sync_copy(enew_hbm.at[pl.ds(off, C)], rows_v)

            def rloop(j, carry2):
                v = idx_v[pl.ds(j * 16, 16)] - lo
                ok = (v >= 0) & (v < HALF)
                idx2_v[pl.ds(j * 16, 16)] = jnp.where(ok, v, HALF)
                return carry2

            lax.fori_loop(0, C // 16, rloop, 0)
            pltpu.sync_copy(rows_v, acc_sh.at[idx2_v], add=True)
            return carry

        lax.fori_loop(0, NCH, body, 0)
        plsc.subcore_barrier()
        pltpu.sync_copy(acc_sh.at[pl.ds(sid * STRIPE, STRIPE)],
                        acc_out.at[pl.ds(cid * HALF + sid * STRIPE, STRIPE)])

    return scatter_kernel(e_new, dst, zeros_pad)


def _ln(x, g, b):
    m = jnp.mean(x, axis=-1, keepdims=True)
    v = jnp.mean((x - m) ** 2, axis=-1, keepdims=True)
    return (x - m) * lax.rsqrt(v + 1e-5) * g + b


def _dot(a, b):
    return jnp.dot(a, b, preferred_element_type=jnp.float32,
                   precision=lax.Precision.DEFAULT)


def _edge_call(e_h, x_s, x_d, p):
    E, D = e_h.shape
    B = 2000
    G = E // B
    W1 = p['W1']
    w1e = W1[:D]
    w1s = W1[D:2 * D]
    w1d = W1[2 * D:]
    vec = lambda v: v.reshape(1, -1)

    def body(eh, xs, xd, rw1e, rw1s, rw1d, rb1, rg1, rbe1, rw2, rb2, rg2, rbe2,
             eout, enew):
        x = eh[...]
        h = (_dot(x, rw1e[...]) + _dot(xs[...], rw1s[...])
             + _dot(xd[...], rw1d[...]) + rb1[...])
        h = _ln(h, rg1[...], rbe1[...])
        h = h * jax.nn.sigmoid(h)
        o = _dot(h, rw2[...]) + rb2[...]
        o = _ln(o, rg2[...], rbe2[...])
        enew[...] = o
        eout[...] = x + o

    row = pl.BlockSpec((B, D), lambda i: (i, 0))
    full = lambda a: pl.BlockSpec(a.shape, lambda i: (0,) * a.ndim)
    args = [w1e, w1s, w1d, vec(p['b1']), vec(p['g1']), vec(p['be1']),
            p['W2'], vec(p['b2']), vec(p['g2']), vec(p['be2'])]
    return pl.pallas_call(
        body,
        grid=(G,),
        in_specs=[row, row, row] + [full(a) for a in args],
        out_specs=[row, row],
        out_shape=[jax.ShapeDtypeStruct((E, D), jnp.float32),
                   jax.ShapeDtypeStruct((E, D), jnp.float32)],
    )(e_h, x_s, x_d, *args)


def _node_call(h_src, h_dst, m, ps, pd):
    N, D = h_src.shape
    B = 1000
    G = N // B
    wd1h, wd1m = pd['W1'][:D], pd['W1'][D:]
    vec = lambda v: v.reshape(1, -1)

    def body(hs, hd, mr,
             sw1, sb1, sg1, sbe1, sw2, sb2, sg2, sbe2,
             dw1h, dw1m, db1, dg1, dbe1, dw2, db2, dg2, dbe2,
             hs_out, hd_out):
        x = hs[...]
        h = _dot(x, sw1[...]) + sb1[...]
        h = _ln(h, sg1[...], sbe1[...])
        h = h * jax.nn.sigmoid(h)
        o = _dot(h, sw2[...]) + sb2[...]
        o = _ln(o, sg2[...], sbe2[...])
        hs_out[...] = x + o

        xd = hd[...]
        h2 = _dot(xd, dw1h[...]) + _dot(mr[...], dw1m[...]) + db1[...]
        h2 = _ln(h2, dg1[...], dbe1[...])
        h2 = h2 * jax.nn.sigmoid(h2)
        o2 = _dot(h2, dw2[...]) + db2[...]
        o2 = _ln(o2, dg2[...], dbe2[...])
        hd_out[...] = xd + o2

    row = pl.BlockSpec((B, D), lambda i: (i, 0))
    full = lambda a: pl.BlockSpec(a.shape, lambda i: (0,) * a.ndim)
    args = [ps['W1'], vec(ps['b1']), vec(ps['g1']), vec(ps['be1']),
            ps['W2'], vec(ps['b2']), vec(ps['g2']), vec(ps['be2']),
            wd1h, wd1m, vec(pd['b1']), vec(pd['g1']), vec(pd['be1']),
            pd['W2'], vec(pd['b2']), vec(pd['g2']), vec(pd['be2'])]
    return pl.pallas_call(
        body,
        grid=(G,),
        in_specs=[row, row, row] + [full(a) for a in args],
        out_specs=[row, row],
        out_shape=[jax.ShapeDtypeStruct((N, D), jnp.float32),
                   jax.ShapeDtypeStruct((N, D), jnp.float32)],
    )(h_src, h_dst, m, *args)


def kernel(h_src, h_dst, e_h, edge_index, params):
    src = edge_index[0].astype(jnp.int32)
    dst = edge_index[1].astype(jnp.int32)
    N, D = h_src.shape
    E = e_h.shape[0]
    xs, xd = _sc_gather(h_src, h_dst, src, dst)
    e_out, e_new = _edge_call(e_h, xs, xd, params['edge'])
    zeros = jnp.zeros((ACC_ROWS, e_h.shape[1]), jnp.float32)
    m = _sc_scatter(e_new, dst, zeros)
    h_src_out, h_dst_out = _node_call(h_src, h_dst, m,
                                      params['src'], params['dst'])
    return (h_src_out, h_dst_out, e_out)


# speedup vs baseline: 3.2470x; 1.0152x over previous
import functools

import jax
import jax.numpy as jnp
from jax import lax
from jax.experimental import pallas as pl
from jax.experimental.pallas import tpu as pltpu
from jax.experimental.pallas import tpu_sc as plsc

NC = 2
NS = 16
NW = NC * NS
CHUNK = 400
NPAD = 10240


def _sc_mesh():
    return plsc.VectorSubcoreMesh(
        core_axis_name="c", subcore_axis_name="s", num_cores=NC, num_subcores=NS
    )


def _sc_gather(h_src, h_dst, src, dst):
    N, D = h_src.shape
    E = src.shape[0]
    EW = E // NW
    C = 200
    NCH = EW // C

    @functools.partial(
        pl.kernel,
        out_type=[jax.ShapeDtypeStruct((E, D), jnp.float32),
                  jax.ShapeDtypeStruct((E, D), jnp.float32)],
        mesh=_sc_mesh(),
        scratch_types=[
            pltpu.VMEM((C,), jnp.int32),
            pltpu.VMEM((C,), jnp.int32),
            pltpu.VMEM((C,), jnp.int32),
            pltpu.VMEM((C,), jnp.int32),
            pltpu.VMEM((C, D), jnp.float32),
            pltpu.VMEM((C, D), jnp.float32),
            pltpu.VMEM((C, D), jnp.float32),
            pltpu.VMEM((C, D), jnp.float32),
            pltpu.SemaphoreType.DMA((12,)),
        ],
    )
    def gather_kernel(hs_hbm, hd_hbm, src_hbm, dst_hbm, out_s, out_d,
                      ibs0, ibs1, ibd0, ibd1, rs0, rs1, rd0, rd1, sem):
        wid = lax.axis_index("s") * NC + lax.axis_index("c")
        base = wid * EW
        ibs = (ibs0, ibs1)
        ibd = (ibd0, ibd1)
        rs = (rs0, rs1)
        rd = (rd0, rd1)
        si_s = (sem.at[0], sem.at[1])
        si_d = (sem.at[2], sem.at[3])
        sg_s = (sem.at[4], sem.at[5])
        sg_d = (sem.at[6], sem.at[7])
        sw_s = (sem.at[8], sem.at[9])
        sw_d = (sem.at[10], sem.at[11])

        def w_start(i, b):
            off = base + i * C
            cs = pltpu.async_copy(rs[b], out_s.at[pl.ds(off, C)], sw_s[b])
            cd = pltpu.async_copy(rd[b], out_d.at[pl.ds(off, C)], sw_d[b])
            return (cs, cd)

        def wait_all(cs):
            for c in cs:
                c.wait()

        def ix_start(i, b):
            off = base + i * C
            cs = pltpu.async_copy(src_hbm.at[pl.ds(off, C)], ibs[b], si_s[b])
            cd = pltpu.async_copy(dst_hbm.at[pl.ds(off, C)], ibd[b], si_d[b])
            return (cs, cd)

        def g_start(b):
            cs = pltpu.async_copy(hs_hbm.at[ibs[b]], rs[b], sg_s[b])
            cd = pltpu.async_copy(hd_hbm.at[ibd[b]], rd[b], sg_d[b])
            return (cs, cd)

        wait_all(ix_start(0, 0))
        wait_all(ix_start(1, 1))
        g_pend = [g_start(0), g_start(1)]
        w_pend = [None, None]
        for i in range(NCH):
            b = i % 2
            wait_all(g_pend[b])
            w_pend[b] = w_start(i, b)
            if i + 2 < NCH:
                ix = ix_start(i + 2, b)
                wait_all(w_pend[b])
                wait_all(ix)
                g_pend[b] = g_start(b)
        wait_all(w_pend[0])
        wait_all(w_pend[1])

    return gather_kernel(h_src, h_dst, src, dst)


HALF = 5120
ACC_ROWS = 5248


def _sc_scatter(e_new, dst, zeros_pad):
    E, D = e_new.shape
    ET = E // NS
    C = 400
    NCH = ET // C
    STRIPE = HALF // NS

    @functools.partial(
        pl.kernel,
        out_type=jax.ShapeDtypeStruct((NC * HALF, D), jnp.float32),
        mesh=_sc_mesh(),
        scratch_types=[
            pltpu.VMEM((C,), jnp.int32),
            pltpu.VMEM((C,), jnp.int32),
            pltpu.VMEM((C, D), jnp.float32),
            pltpu.VMEM_SHARED((ACC_ROWS, D), jnp.float32),
            pltpu.SemaphoreType.DMA((2,)),
        ],
    )
    def scatter_kernel(enew_hbm, dst_hbm, zeros_hbm, acc_out,
                       idx_v, idx2_v, rows_v, acc_sh, sem):
        cid = lax.axis_index("c")
        sid = lax.axis_index("s")
        zst = ACC_ROWS // NS
        pltpu.sync_copy(zeros_hbm.at[pl.ds(sid * zst, zst)],
                        acc_sh.at[pl.ds(sid * zst, zst)])
        plsc.subcore_barrier()
        base = sid * ET
        lo = cid * HALF

        def body(i, carry):
            off = base + i * C
            pltpu.sync_copy(dst_hbm.at[pl.ds(off, C)], idx_v)
            pltpu.sync_copy(enew_hbm.at[pl.ds(off, C)], rows_v)

            def rloop(j, carry2):
                v = idx_v[pl.ds(j * 16, 16)] - lo
                ok = (v >= 0) & (v < HALF)
                idx2_v[pl.ds(j * 16, 16)] = jnp.where(ok, v, HALF)
                return carry2

            lax.fori_loop(0, C // 16, rloop, 0)
            pltpu.sync_copy(rows_v, acc_sh.at[idx2_v], add=True)
            return carry

        lax.fori_loop(0, NCH, body, 0)
        plsc.subcore_barrier()
        pltpu.sync_copy(acc_sh.at[pl.ds(sid * STRIPE, STRIPE)],
                        acc_out.at[pl.ds(cid * HALF + sid * STRIPE, STRIPE)])

    return scatter_kernel(e_new, dst, zeros_pad)


def _ln(x, g, b):
    m = jnp.mean(x, axis=-1, keepdims=True)
    v = jnp.mean((x - m) ** 2, axis=-1, keepdims=True)
    return (x - m) * lax.rsqrt(v + 1e-5) * g + b


def _dot(a, b):
    return jnp.dot(a, b, preferred_element_type=jnp.float32,
                   precision=lax.Precision.DEFAULT)


def _edge_call(e_h, x_s, x_d, p):
    E, D = e_h.shape
    B = 4000
    G = E // B
    W1 = p['W1']
    w1e = W1[:D]
    w1s = W1[D:2 * D]
    w1d = W1[2 * D:]
    vec = lambda v: v.reshape(1, -1)

    def body(eh, xs, xd, rw1e, rw1s, rw1d, rb1, rg1, rbe1, rw2, rb2, rg2, rbe2,
             eout, enew):
        x = eh[...]
        h = (_dot(x, rw1e[...]) + _dot(xs[...], rw1s[...])
             + _dot(xd[...], rw1d[...]) + rb1[...])
        h = _ln(h, rg1[...], rbe1[...])
        h = h * jax.nn.sigmoid(h)
        o = _dot(h, rw2[...]) + rb2[...]
        o = _ln(o, rg2[...], rbe2[...])
        enew[...] = o
        eout[...] = x + o

    row = pl.BlockSpec((B, D), lambda i: (i, 0))
    full = lambda a: pl.BlockSpec(a.shape, lambda i: (0,) * a.ndim)
    args = [w1e, w1s, w1d, vec(p['b1']), vec(p['g1']), vec(p['be1']),
            p['W2'], vec(p['b2']), vec(p['g2']), vec(p['be2'])]
    return pl.pallas_call(
        body,
        grid=(G,),
        in_specs=[row, row, row] + [full(a) for a in args],
        out_specs=[row, row],
        out_shape=[jax.ShapeDtypeStruct((E, D), jnp.float32),
                   jax.ShapeDtypeStruct((E, D), jnp.float32)],
    )(e_h, x_s, x_d, *args)


def _node_call(h_src, h_dst, m, ps, pd):
    N, D = h_src.shape
    B = 1000
    G = N // B
    wd1h, wd1m = pd['W1'][:D], pd['W1'][D:]
    vec = lambda v: v.reshape(1, -1)

    def body(hs, hd, mr,
             sw1, sb1, sg1, sbe1, sw2, sb2, sg2, sbe2,
             dw1h, dw1m, db1, dg1, dbe1, dw2, db2, dg2, dbe2,
             hs_out, hd_out):
        x = hs[...]
        h = _dot(x, sw1[...]) + sb1[...]
        h = _ln(h, sg1[...], sbe1[...])
        h = h * jax.nn.sigmoid(h)
        o = _dot(h, sw2[...]) + sb2[...]
        o = _ln(o, sg2[...], sbe2[...])
        hs_out[...] = x + o

        xd = hd[...]
        h2 = _dot(xd, dw1h[...]) + _dot(mr[...], dw1m[...]) + db1[...]
        h2 = _ln(h2, dg1[...], dbe1[...])
        h2 = h2 * jax.nn.sigmoid(h2)
        o2 = _dot(h2, dw2[...]) + db2[...]
        o2 = _ln(o2, dg2[...], dbe2[...])
        hd_out[...] = xd + o2

    row = pl.BlockSpec((B, D), lambda i: (i, 0))
    full = lambda a: pl.BlockSpec(a.shape, lambda i: (0,) * a.ndim)
    args = [ps['W1'], vec(ps['b1']), vec(ps['g1']), vec(ps['be1']),
            ps['W2'], vec(ps['b2']), vec(ps['g2']), vec(ps['be2']),
            wd1h, wd1m, vec(pd['b1']), vec(pd['g1']), vec(pd['be1']),
            pd['W2'], vec(pd['b2']), vec(pd['g2']), vec(pd['be2'])]
    return pl.pallas_call(
        body,
        grid=(G,),
        in_specs=[row, row, row] + [full(a) for a in args],
        out_specs=[row, row],
        out_shape=[jax.ShapeDtypeStruct((N, D), jnp.float32),
                   jax.ShapeDtypeStruct((N, D), jnp.float32)],
    )(h_src, h_dst, m, *args)


def kernel(h_src, h_dst, e_h, edge_index, params):
    src = edge_index[0].astype(jnp.int32)
    dst = edge_index[1].astype(jnp.int32)
    N, D = h_src.shape
    E = e_h.shape[0]
    xs, xd = _sc_gather(h_src, h_dst, src, dst)
    e_out, e_new = _edge_call(e_h, xs, xd, params['edge'])
    zeros = jnp.zeros((ACC_ROWS, e_h.shape[1]), jnp.float32)
    m = _sc_scatter(e_new, dst, zeros)
    h_src_out, h_dst_out = _node_call(h_src, h_dst, m,
                                      params['src'], params['dst'])
    return (h_src_out, h_dst_out, e_out)


# speedup vs baseline: 3.5290x; 1.0868x over previous
import functools

import jax
import jax.numpy as jnp
from jax import lax
from jax.experimental import pallas as pl
from jax.experimental.pallas import tpu as pltpu
from jax.experimental.pallas import tpu_sc as plsc

NC = 2
NS = 16
NW = NC * NS
CHUNK = 400
NPAD = 10240


def _sc_mesh():
    return plsc.VectorSubcoreMesh(
        core_axis_name="c", subcore_axis_name="s", num_cores=NC, num_subcores=NS
    )


def _sc_gather(h_src, h_dst, src, dst):
    N, D = h_src.shape
    E = src.shape[0]
    EW = E // NW
    C = 200
    NCH = EW // C

    @functools.partial(
        pl.kernel,
        out_type=[jax.ShapeDtypeStruct((E, D), jnp.float32),
                  jax.ShapeDtypeStruct((E, D), jnp.float32)],
        mesh=_sc_mesh(),
        scratch_types=[
            pltpu.VMEM((C,), jnp.int32),
            pltpu.VMEM((C,), jnp.int32),
            pltpu.VMEM((C,), jnp.int32),
            pltpu.VMEM((C,), jnp.int32),
            pltpu.VMEM((C, D), jnp.float32),
            pltpu.VMEM((C, D), jnp.float32),
            pltpu.VMEM((C, D), jnp.float32),
            pltpu.VMEM((C, D), jnp.float32),
            pltpu.SemaphoreType.DMA((12,)),
        ],
    )
    def gather_kernel(hs_hbm, hd_hbm, src_hbm, dst_hbm, out_s, out_d,
                      ibs0, ibs1, ibd0, ibd1, rs0, rs1, rd0, rd1, sem):
        wid = lax.axis_index("s") * NC + lax.axis_index("c")
        base = wid * EW
        ibs = (ibs0, ibs1)
        ibd = (ibd0, ibd1)
        rs = (rs0, rs1)
        rd = (rd0, rd1)
        si_s = (sem.at[0], sem.at[1])
        si_d = (sem.at[2], sem.at[3])
        sg_s = (sem.at[4], sem.at[5])
        sg_d = (sem.at[6], sem.at[7])
        sw_s = (sem.at[8], sem.at[9])
        sw_d = (sem.at[10], sem.at[11])

        def w_start(i, b):
            off = base + i * C
            cs = pltpu.async_copy(rs[b], out_s.at[pl.ds(off, C)], sw_s[b])
            cd = pltpu.async_copy(rd[b], out_d.at[pl.ds(off, C)], sw_d[b])
            return (cs, cd)

        def wait_all(cs):
            for c in cs:
                c.wait()

        def ix_start(i, b):
            off = base + i * C
            cs = pltpu.async_copy(src_hbm.at[pl.ds(off, C)], ibs[b], si_s[b])
            cd = pltpu.async_copy(dst_hbm.at[pl.ds(off, C)], ibd[b], si_d[b])
            return (cs, cd)

        def g_start(b):
            cs = pltpu.async_copy(hs_hbm.at[ibs[b]], rs[b], sg_s[b])
            cd = pltpu.async_copy(hd_hbm.at[ibd[b]], rd[b], sg_d[b])
            return (cs, cd)

        wait_all(ix_start(0, 0))
        wait_all(ix_start(1, 1))
        g_pend = [g_start(0), g_start(1)]
        w_pend = [None, None]
        for i in range(NCH):
            b = i % 2
            wait_all(g_pend[b])
            w_pend[b] = w_start(i, b)
            if i + 2 < NCH:
                ix = ix_start(i + 2, b)
                wait_all(w_pend[b])
                wait_all(ix)
                g_pend[b] = g_start(b)
        wait_all(w_pend[0])
        wait_all(w_pend[1])

    return gather_kernel(h_src, h_dst, src, dst)


HALF = 5120
ACC_ROWS = 5248


def _sc_scatter(e_new, dst, zeros_pad):
    E, D = e_new.shape
    ET = E // NS
    C = 400
    NCH = ET // C
    STRIPE = HALF // NS

    @functools.partial(
        pl.kernel,
        out_type=jax.ShapeDtypeStruct((NC * HALF, D), jnp.float32),
        mesh=_sc_mesh(),
        scratch_types=[
            pltpu.VMEM((C,), jnp.int32),
            pltpu.VMEM((C,), jnp.int32),
            pltpu.VMEM((C, D), jnp.float32),
            pltpu.VMEM_SHARED((ACC_ROWS, D), jnp.float32),
            pltpu.SemaphoreType.DMA((2,)),
        ],
    )
    def scatter_kernel(enew_hbm, dst_hbm, zeros_hbm, acc_out,
                       idx_v, idx2_v, rows_v, acc_sh, sem):
        cid = lax.axis_index("c")
        sid = lax.axis_index("s")
        zst = ACC_ROWS // NS
        pltpu.sync_copy(zeros_hbm.at[pl.ds(sid * zst, zst)],
                        acc_sh.at[pl.ds(sid * zst, zst)])
        plsc.subcore_barrier()
        base = sid * ET
        lo = cid * HALF

        def body(i, carry):
            off = base + i * C
            pltpu.sync_copy(dst_hbm.at[pl.ds(off, C)], idx_v)
            pltpu.sync_copy(enew_hbm.at[pl.ds(off, C)], rows_v)

            def rloop(j, carry2):
                v = idx_v[pl.ds(j * 16, 16)] - lo
                ok = (v >= 0) & (v < HALF)
                idx2_v[pl.ds(j * 16, 16)] = jnp.where(ok, v, HALF)
                return carry2

            lax.fori_loop(0, C // 16, rloop, 0)
            pltpu.sync_copy(rows_v, acc_sh.at[idx2_v], add=True)
            return carry

        lax.fori_loop(0, NCH, body, 0)
        plsc.subcore_barrier()
        pltpu.sync_copy(acc_sh.at[pl.ds(sid * STRIPE, STRIPE)],
                        acc_out.at[pl.ds(cid * HALF + sid * STRIPE, STRIPE)])

    return scatter_kernel(e_new, dst, zeros_pad)


def _ln(x, g, b):
    m = jnp.mean(x, axis=-1, keepdims=True)
    v = jnp.mean((x - m) ** 2, axis=-1, keepdims=True)
    return (x - m) * lax.rsqrt(v + 1e-5) * g + b


def _dot(a, b):
    return jnp.dot(a, b, preferred_element_type=jnp.float32,
                   precision=lax.Precision.DEFAULT)


def _edge_call(e_h, x_s, x_d, p):
    E, D = e_h.shape
    B = 4000
    G = E // B
    W1 = p['W1']
    w1e = W1[:D]
    w1s = W1[D:2 * D]
    w1d = W1[2 * D:]
    vec = lambda v: v.reshape(1, -1)

    def body(eh, xs, xd, rw1e, rw1s, rw1d, rb1, rg1, rbe1, rw2, rb2, rg2, rbe2,
             eout, enew):
        x = eh[...]
        h = (_dot(x, rw1e[...]) + _dot(xs[...], rw1s[...])
             + _dot(xd[...], rw1d[...]) + rb1[...])
        h = _ln(h, rg1[...], rbe1[...])
        h = h * jax.nn.sigmoid(h)
        o = _dot(h, rw2[...]) + rb2[...]
        o = _ln(o, rg2[...], rbe2[...])
        enew[...] = o
        eout[...] = x + o

    row = pl.BlockSpec((B, D), lambda i: (i, 0))
    full = lambda a: pl.BlockSpec(a.shape, lambda i: (0,) * a.ndim)
    args = [w1e, w1s, w1d, vec(p['b1']), vec(p['g1']), vec(p['be1']),
            p['W2'], vec(p['b2']), vec(p['g2']), vec(p['be2'])]
    return pl.pallas_call(
        body,
        grid=(G,),
        in_specs=[row, row, row] + [full(a) for a in args],
        out_specs=[row, row],
        out_shape=[jax.ShapeDtypeStruct((E, D), jnp.float32),
                   jax.ShapeDtypeStruct((E, D), jnp.float32)],
    )(e_h, x_s, x_d, *args)


def _node_call(h_src, h_dst, m0, m1, ps, pd):
    N, D = h_src.shape
    B = 1000
    G = N // B
    wd1h, wd1m = pd['W1'][:D], pd['W1'][D:]
    vec = lambda v: v.reshape(1, -1)

    def body(hs, hd, m0r, m1r,
             sw1, sb1, sg1, sbe1, sw2, sb2, sg2, sbe2,
             dw1h, dw1m, db1, dg1, dbe1, dw2, db2, dg2, dbe2,
             hs_out, hd_out):
        x = hs[...]
        h = _dot(x, sw1[...]) + sb1[...]
        h = _ln(h, sg1[...], sbe1[...])
        h = h * jax.nn.sigmoid(h)
        o = _dot(h, sw2[...]) + sb2[...]
        o = _ln(o, sg2[...], sbe2[...])
        hs_out[...] = x + o

        xd = hd[...]
        h2 = _dot(xd, dw1h[...]) + _dot(m0r[...] + m1r[...], dw1m[...]) + db1[...]
        h2 = _ln(h2, dg1[...], dbe1[...])
        h2 = h2 * jax.nn.sigmoid(h2)
        o2 = _dot(h2, dw2[...]) + db2[...]
        o2 = _ln(o2, dg2[...], dbe2[...])
        hd_out[...] = xd + o2

    row = pl.BlockSpec((B, D), lambda i: (i, 0))
    full = lambda a: pl.BlockSpec(a.shape, lambda i: (0,) * a.ndim)
    args = [ps['W1'], vec(ps['b1']), vec(ps['g1']), vec(ps['be1']),
            ps['W2'], vec(ps['b2']), vec(ps['g2']), vec(ps['be2']),
            wd1h, wd1m, vec(pd['b1']), vec(pd['g1']), vec(pd['be1']),
            pd['W2'], vec(pd['b2']), vec(pd['g2']), vec(pd['be2'])]
    return pl.pallas_call(
        body,
        grid=(G,),
        in_specs=[row, row, row, row] + [full(a) for a in args],
        out_specs=[row, row],
        out_shape=[jax.ShapeDtypeStruct((N, D), jnp.float32),
                   jax.ShapeDtypeStruct((N, D), jnp.float32)],
    )(h_src, h_dst, m0, m1, *args)


def kernel(h_src, h_dst, e_h, edge_index, params):
    src = edge_index[0].astype(jnp.int32)
    dst = edge_index[1].astype(jnp.int32)
    N, D = h_src.shape
    E = e_h.shape[0]
    zeros = jnp.zeros((ACC_ROWS, D), jnp.float32)
    Eh = E // 2
    src1, src2 = src[:Eh], src[Eh:]
    dst1, dst2 = dst[:Eh], dst[Eh:]
    xs1, xd1 = _sc_gather(h_src, h_dst, src1, dst1)
    src2b, _ = lax.optimization_barrier((src2, xs1[0, 0]))
    xs2, xd2 = _sc_gather(h_src, h_dst, src2b, dst2)
    e_out1, e_new1 = _edge_call(e_h[:Eh], xs1, xd1, params['edge'])
    dst1b, _ = lax.optimization_barrier((dst1, xs2[0, 0]))
    m1 = _sc_scatter(e_new1, dst1b, zeros)
    e_out2, e_new2 = _edge_call(e_h[Eh:], xs2, xd2, params['edge'])
    dst2b, _ = lax.optimization_barrier((dst2, m1[0, 0]))
    m2 = _sc_scatter(e_new2, dst2b, zeros)
    e_out = jnp.concatenate([e_out1, e_out2], axis=0)
    h_src_out, h_dst_out = _node_call(h_src, h_dst, m1, m2,
                                      params['src'], params['dst'])
    return (h_src_out, h_dst_out, e_out)
